# baseline (speedup 1.0000x reference)
import functools

import jax
import jax.numpy as jnp
from jax import lax
from jax.experimental import pallas as pl
from jax.experimental.pallas import tpu as pltpu
from jax.experimental.pallas import tpu_sc as plsc

N = 10000
D = 128
H = D // 2
C = 256
SG = 1
NS = 16
NBUF = 2
NP = 10240
ROWS_PER_TILE = NP // NS
ROWS_PER_COPY = ROWS_PER_TILE // 5
UNIT_E = SG * C


def _prep_edges(edge_index, edge_weight, units_per_tile):
    e = edge_index.shape[1]
    ep = NS * units_per_tile * UNIT_E
    col = jnp.pad(edge_index[1].astype(jnp.int32), (0, ep - e))
    row = jnp.pad(edge_index[0].astype(jnp.int32), (0, ep - e))
    w = jnp.pad(edge_weight, (0, ep - e))
    nu = ep // UNIT_E
    col = col.reshape(nu, C)
    row = row.reshape(nu, C)
    wb = jax.lax.bitcast_convert_type(w, jnp.int32).reshape(nu, C)
    core0 = jnp.stack([col, row, wb], axis=1)
    core1 = jnp.stack([col + N, row, wb], axis=1)
    icrw = jnp.stack([core0, core1])
    return jnp.pad(icrw, ((0, 0), (0, 2 * NBUF), (0, 0), (0, 0)))


def _sc_body(units1, units2,
             xcat_h, icrw1_h, icrw2_h, out_h,
             cb, gx, out1_sh, out2_sh, *sems):
    c = lax.axis_index("c")
    s = lax.axis_index("s")
    isems = list(sems[:2])
    gsems = list(sems[2:2 + NBUF])
    ssems = list(sems[2 + NBUF:2 + 2 * NBUF])

    zv = jnp.zeros((16,), jnp.float32)

    @pl.loop(0, ROWS_PER_COPY)
    def _zero(e):
        for j in range(H // 16):
            gx[0, e, pl.ds(j * 16, 16)] = zv

    base_row = s * ROWS_PER_TILE
    for k in range(5):
        r = base_row + k * ROWS_PER_COPY
        pltpu.sync_copy(gx.at[0, pl.ds(0, ROWS_PER_COPY)],
                        out1_sh.at[pl.ds(r, ROWS_PER_COPY)])
        pltpu.sync_copy(gx.at[0, pl.ds(0, ROWS_PER_COPY)],
                        out2_sh.at[pl.ds(r, ROWS_PER_COPY)])
    plsc.subcore_barrier()

    def do_edges(icrw_h, out_sh, units_per_tile):
        ubase = s * units_per_tile
        n_groups = units_per_tile // NBUF

        def idx_prefetch(slot, g):
            base = ubase + g * NBUF
            pltpu.async_copy(icrw_h.at[c, pl.ds(base, NBUF)], cb.at[slot],
                             isems[slot])

        def idx_wait(slot):
            pltpu.make_async_copy(icrw_h.at[c, pl.ds(ubase, NBUF)],
                                  cb.at[slot], isems[slot]).wait()

        def gather(slot, b):
            pltpu.async_copy(xcat_h.at[cb.at[slot, b, 0]], gx.at[b], gsems[b])

        def gather_wait(b):
            pltpu.make_async_copy(xcat_h.at[cb.at[0, b, 0]], gx.at[b],
                                  gsems[b]).wait()

        def scatter(slot, b):
            pltpu.async_copy(gx.at[b], out_sh.at[cb.at[slot, b, 1]], ssems[b],
                             add=True)

        def scatter_wait(slot, b):
            pltpu.make_async_copy(gx.at[b], out_sh.at[cb.at[slot, b, 1]],
                                  ssems[b]).wait()

        def scale(slot, b):
            @pl.loop(0, C // 16)
            def _scale(g16):
                wi = cb[slot, b, 2, pl.ds(g16 * 16, 16)]
                wv = plsc.bitcast(wi, jnp.float32)
                for i in range(16):
                    e = g16 * 16 + i
                    w = wv[i]
                    for j in range(H // 16):
                        gx[b, e, pl.ds(j * 16, 16)] = (
                            gx[b, e, pl.ds(j * 16, 16)] * w)

        idx_prefetch(0, 0)
        idx_wait(0)
        idx_prefetch(1, 1)
        for b in range(NBUF):
            gather(0, b)

        @pl.loop(0, n_groups // 2)
        def _grp(gi):
            g = gi * 2
            for ph in range(2):
                other = 1 - ph
                for b in range(NBUF):
                    gather_wait(b)
                    scale(ph, b)
                    scatter(ph, b)
                idx_wait(other)
                for b in range(NBUF):
                    scatter_wait(ph, b)
                    gather(other, b)
                idx_prefetch(ph, g + ph + 2)

        for b in range(NBUF):
            gather_wait(b)
        idx_wait(1)

    do_edges(icrw1_h, out1_sh, units1)
    do_edges(icrw2_h, out2_sh, units2)

    plsc.subcore_barrier()
    for k in range(5):
        r = base_row + k * ROWS_PER_COPY
        pltpu.sync_copy(out1_sh.at[pl.ds(r, ROWS_PER_COPY)],
                        out_h.at[0, c, pl.ds(r, ROWS_PER_COPY)])
        pltpu.sync_copy(out2_sh.at[pl.ds(r, ROWS_PER_COPY)],
                        out_h.at[1, c, pl.ds(r, ROWS_PER_COPY)])


@jax.jit
def kernel(x, edge_index, edge_weight, edge_index2, edge_weight2):
    e1 = edge_index.shape[1]
    e2 = edge_index2.shape[1]
    per = NS * UNIT_E * 2 * NBUF
    units1 = (-(-e1 // per) * per) // (NS * UNIT_E)
    units2 = (-(-e2 // per) * per) // (NS * UNIT_E)

    xcat = jnp.concatenate([x[:, :H], x[:, H:]], axis=0)
    icrw1 = _prep_edges(edge_index, edge_weight, units1)
    icrw2 = _prep_edges(edge_index2, edge_weight2, units2)

    mesh = plsc.VectorSubcoreMesh(core_axis_name="c", subcore_axis_name="s")
    run = pl.kernel(
        functools.partial(_sc_body, units1, units2),
        out_type=jax.ShapeDtypeStruct((2, 2, NP, H), jnp.float32),
        mesh=mesh,
        scratch_types=[
            pltpu.VMEM((2, NBUF, 3, C), jnp.int32),
            pltpu.VMEM((NBUF, C, H), jnp.float32),
            pltpu.VMEM_SHARED((NP, H), jnp.float32),
            pltpu.VMEM_SHARED((NP, H), jnp.float32),
        ] + [pltpu.SemaphoreType.DMA] * (2 + 2 * NBUF),
        compiler_params=pltpu.CompilerParams(use_tc_tiling_on_sc=False,
                                             needs_layout_passes=False),
        name="h2gcn_spmm_sc",
    )
    out = run(xcat, icrw1, icrw2)
    return jnp.concatenate([out[0, 0, :N], out[0, 1, :N],
                            out[1, 0, :N], out[1, 1, :N]], axis=1)


# speedup vs baseline: 1.0653x; 1.0653x over previous
import functools

import jax
import jax.numpy as jnp
from jax import lax
from jax.experimental import pallas as pl
from jax.experimental.pallas import tpu as pltpu
from jax.experimental.pallas import tpu_sc as plsc

N = 10000
D = 128
H = D // 2
C = 256
SG = 1
NS = 16
NBUF = 4
NP = 10240
ROWS_PER_TILE = NP // NS
ROWS_PER_COPY = ROWS_PER_TILE // 5
UNIT_E = SG * C


def _prep_edges(edge_index, edge_weight, units_per_tile):
    e = edge_index.shape[1]
    ep = NS * units_per_tile * UNIT_E
    col = jnp.pad(edge_index[1].astype(jnp.int32), (0, ep - e))
    row = jnp.pad(edge_index[0].astype(jnp.int32), (0, ep - e))
    w = jnp.pad(edge_weight, (0, ep - e))
    nu = ep // UNIT_E
    col = col.reshape(nu, C)
    row = row.reshape(nu, C)
    wb = jax.lax.bitcast_convert_type(w, jnp.int32).reshape(nu, C)
    core0 = jnp.stack([col, row, wb], axis=1)
    core1 = jnp.stack([col + N, row, wb], axis=1)
    icrw = jnp.stack([core0, core1])
    return jnp.pad(icrw, ((0, 0), (0, 2 * NBUF), (0, 0), (0, 0)))


def _sc_body(units1, units2,
             xcat_h, icrw1_h, icrw2_h, out_h,
             cb, gx, out_sh, *sems):
    c = lax.axis_index("c")
    s = lax.axis_index("s")
    isems = list(sems[:2])
    gsems = list(sems[2:2 + NBUF])
    ssems = list(sems[2 + NBUF:2 + 2 * NBUF])

    base_row = s * ROWS_PER_TILE
    zv = jnp.zeros((16,), jnp.float32)

    def zero_accumulator():
        @pl.loop(0, ROWS_PER_COPY)
        def _zero(e):
            for j in range(H // 16):
                gx[0, e, pl.ds(j * 16, 16)] = zv
        for k in range(5):
            r = base_row + k * ROWS_PER_COPY
            pltpu.sync_copy(gx.at[0, pl.ds(0, ROWS_PER_COPY)],
                            out_sh.at[pl.ds(r, ROWS_PER_COPY)])

    def copy_out(which):
        for k in range(5):
            r = base_row + k * ROWS_PER_COPY
            pltpu.sync_copy(out_sh.at[pl.ds(r, ROWS_PER_COPY)],
                            out_h.at[which, c, pl.ds(r, ROWS_PER_COPY)])

    def do_edges(icrw_h, units_per_tile):
        ubase = s * units_per_tile
        n_groups = units_per_tile // NBUF

        def idx_prefetch(slot, g):
            base = ubase + g * NBUF
            pltpu.async_copy(icrw_h.at[c, pl.ds(base, NBUF)], cb.at[slot],
                             isems[slot])

        def idx_wait(slot):
            pltpu.make_async_copy(icrw_h.at[c, pl.ds(ubase, NBUF)],
                                  cb.at[slot], isems[slot]).wait()

        def gather(slot, b):
            pltpu.async_copy(xcat_h.at[cb.at[slot, b, 0]], gx.at[b], gsems[b])

        def gather_wait(b):
            pltpu.make_async_copy(xcat_h.at[cb.at[0, b, 0]], gx.at[b],
                                  gsems[b]).wait()

        def scatter(slot, b):
            pltpu.async_copy(gx.at[b], out_sh.at[cb.at[slot, b, 1]], ssems[b],
                             add=True)

        def scatter_wait(slot, b):
            pltpu.make_async_copy(gx.at[b], out_sh.at[cb.at[slot, b, 1]],
                                  ssems[b]).wait()

        def scale(slot, b):
            @pl.loop(0, C // 16)
            def _scale(g16):
                wi = cb[slot, b, 2, pl.ds(g16 * 16, 16)]
                wv = plsc.bitcast(wi, jnp.float32)
                for i in range(16):
                    e = g16 * 16 + i
                    w = wv[i]
                    for j in range(H // 16):
                        gx[b, e, pl.ds(j * 16, 16)] = (
                            gx[b, e, pl.ds(j * 16, 16)] * w)

        idx_prefetch(0, 0)
        idx_wait(0)
        idx_prefetch(1, 1)
        for b in range(NBUF):
            gather(0, b)

        @pl.loop(0, n_groups // 2)
        def _grp(gi):
            g = gi * 2
            for ph in range(2):
                other = 1 - ph
                for b in range(NBUF):
                    gather_wait(b)
                    scale(ph, b)
                    scatter(ph, b)
                idx_wait(other)
                for b in range(NBUF):
                    scatter_wait(ph, b)
                    gather(other, b)
                idx_prefetch(ph, g + ph + 2)

        for b in range(NBUF):
            gather_wait(b)
        idx_wait(1)

    zero_accumulator()
    plsc.subcore_barrier()
    do_edges(icrw1_h, units1)
    plsc.subcore_barrier()
    copy_out(0)
    zero_accumulator()
    plsc.subcore_barrier()
    do_edges(icrw2_h, units2)
    plsc.subcore_barrier()
    copy_out(1)


@jax.jit
def kernel(x, edge_index, edge_weight, edge_index2, edge_weight2):
    e1 = edge_index.shape[1]
    e2 = edge_index2.shape[1]
    per = NS * UNIT_E * 2 * NBUF
    units1 = (-(-e1 // per) * per) // (NS * UNIT_E)
    units2 = (-(-e2 // per) * per) // (NS * UNIT_E)

    xcat = jnp.concatenate([x[:, :H], x[:, H:]], axis=0)
    icrw1 = _prep_edges(edge_index, edge_weight, units1)
    icrw2 = _prep_edges(edge_index2, edge_weight2, units2)

    mesh = plsc.VectorSubcoreMesh(core_axis_name="c", subcore_axis_name="s")
    run = pl.kernel(
        functools.partial(_sc_body, units1, units2),
        out_type=jax.ShapeDtypeStruct((2, 2, NP, H), jnp.float32),
        mesh=mesh,
        scratch_types=[
            pltpu.VMEM((2, NBUF, 3, C), jnp.int32),
            pltpu.VMEM((NBUF, C, H), jnp.float32),
            pltpu.VMEM_SHARED((NP, H), jnp.float32),
        ] + [pltpu.SemaphoreType.DMA] * (2 + 2 * NBUF),
        compiler_params=pltpu.CompilerParams(use_tc_tiling_on_sc=False,
                                             needs_layout_passes=False),
        name="h2gcn_spmm_sc",
    )
    out = run(xcat, icrw1, icrw2)
    return jnp.concatenate([out[0, 0, :N], out[0, 1, :N],
                            out[1, 0, :N], out[1, 1, :N]], axis=1)


# speedup vs baseline: 1.0657x; 1.0004x over previous
import functools

import jax
import jax.numpy as jnp
from jax import lax
from jax.experimental import pallas as pl
from jax.experimental.pallas import tpu as pltpu
from jax.experimental.pallas import tpu_sc as plsc

N = 10000
D = 128
H = D // 2
C = 256
SG = 1
NS = 16
NBUF = 4
NP = 10240
ROWS_PER_TILE = NP // NS
ROWS_PER_COPY = ROWS_PER_TILE // 5
UNIT_E = SG * C


def _prep_edges(edge_index, edge_weight, units_per_tile):
    e = edge_index.shape[1]
    ep = NS * units_per_tile * UNIT_E
    col = jnp.pad(edge_index[1].astype(jnp.int32), (0, ep - e))
    row = jnp.pad(edge_index[0].astype(jnp.int32), (0, ep - e))
    w = jnp.pad(edge_weight, (0, ep - e))
    nu = ep // UNIT_E
    col = col.reshape(nu, C)
    row = row.reshape(nu, C)
    wb = jax.lax.bitcast_convert_type(w, jnp.int32).reshape(nu, C)
    core0 = jnp.stack([col, row, wb], axis=1)
    core1 = jnp.stack([col + N, row, wb], axis=1)
    icrw = jnp.stack([core0, core1])
    return jnp.pad(icrw, ((0, 0), (0, 2 * NBUF), (0, 0), (0, 0)))


def _sc_body(units1, units2,
             xcat_h, icrw1_h, icrw2_h, out_h,
             cb, gx, out_sh, *sems):
    c = lax.axis_index("c")
    s = lax.axis_index("s")
    isems = list(sems[:2])
    gsems = list(sems[2:2 + NBUF])
    ssems = list(sems[2 + NBUF:2 + 2 * NBUF])

    base_row = s * ROWS_PER_TILE
    zv = jnp.zeros((16,), jnp.float32)

    def zero_accumulator():
        @pl.loop(0, ROWS_PER_COPY)
        def _zero(e):
            for j in range(H // 16):
                gx[0, e, pl.ds(j * 16, 16)] = zv
        for k in range(5):
            r = base_row + k * ROWS_PER_COPY
            pltpu.sync_copy(gx.at[0, pl.ds(0, ROWS_PER_COPY)],
                            out_sh.at[pl.ds(r, ROWS_PER_COPY)])

    def copy_out(which):
        for k in range(5):
            r = base_row + k * ROWS_PER_COPY
            pltpu.sync_copy(out_sh.at[pl.ds(r, ROWS_PER_COPY)],
                            out_h.at[which, c, pl.ds(r, ROWS_PER_COPY)])

    def do_edges(icrw_h, units_per_tile):
        ubase = s * units_per_tile
        n_groups = units_per_tile // NBUF

        def idx_prefetch(slot, g):
            base = ubase + g * NBUF
            pltpu.async_copy(icrw_h.at[c, pl.ds(base, NBUF)], cb.at[slot],
                             isems[slot])

        def idx_wait(slot):
            pltpu.make_async_copy(icrw_h.at[c, pl.ds(ubase, NBUF)],
                                  cb.at[slot], isems[slot]).wait()

        def gather(slot, b):
            pltpu.async_copy(xcat_h.at[cb.at[slot, b, 0]], gx.at[b], gsems[b])

        def gather_wait(b):
            pltpu.make_async_copy(xcat_h.at[cb.at[0, b, 0]], gx.at[b],
                                  gsems[b]).wait()

        def scatter(slot, b):
            pltpu.async_copy(gx.at[b], out_sh.at[cb.at[slot, b, 1]], ssems[b],
                             add=True)

        def scatter_wait(slot, b):
            pltpu.make_async_copy(gx.at[b], out_sh.at[cb.at[slot, b, 1]],
                                  ssems[b]).wait()

        def scale(slot, b):
            @pl.loop(0, C // 16)
            def _scale(g16):
                wi = cb[slot, b, 2, pl.ds(g16 * 16, 16)]
                wv = plsc.bitcast(wi, jnp.float32)
                for i in range(16):
                    e = g16 * 16 + i
                    w = wv[i]
                    for j in range(H // 16):
                        gx[b, e, pl.ds(j * 16, 16)] = (
                            gx[b, e, pl.ds(j * 16, 16)] * w)

        idx_prefetch(0, 0)
        idx_wait(0)
        idx_prefetch(1, 1)
        for b in range(NBUF):
            gather(0, b)

        @pl.loop(0, n_groups // 2)
        def _grp(gi):
            g = gi * 2
            for ph in range(2):
                other = 1 - ph
                for b in range(NBUF):
                    with jax.named_scope("gwait"):
                        gather_wait(b)
                    with jax.named_scope("scale"):
                        scale(ph, b)
                    with jax.named_scope("sissue"):
                        scatter(ph, b)
                with jax.named_scope("iwait"):
                    idx_wait(other)
                for b in range(NBUF):
                    with jax.named_scope("swait"):
                        scatter_wait(ph, b)
                    with jax.named_scope("gissue"):
                        gather(other, b)
                with jax.named_scope("ipf"):
                    idx_prefetch(ph, g + ph + 2)

        for b in range(NBUF):
            gather_wait(b)
        idx_wait(1)

    zero_accumulator()
    plsc.subcore_barrier()
    do_edges(icrw1_h, units1)
    plsc.subcore_barrier()
    copy_out(0)
    zero_accumulator()
    plsc.subcore_barrier()
    do_edges(icrw2_h, units2)
    plsc.subcore_barrier()
    copy_out(1)


@jax.jit
def kernel(x, edge_index, edge_weight, edge_index2, edge_weight2):
    e1 = edge_index.shape[1]
    e2 = edge_index2.shape[1]
    per = NS * UNIT_E * 2 * NBUF
    units1 = (-(-e1 // per) * per) // (NS * UNIT_E)
    units2 = (-(-e2 // per) * per) // (NS * UNIT_E)

    xcat = jnp.concatenate([x[:, :H], x[:, H:]], axis=0)
    icrw1 = _prep_edges(edge_index, edge_weight, units1)
    icrw2 = _prep_edges(edge_index2, edge_weight2, units2)

    mesh = plsc.VectorSubcoreMesh(core_axis_name="c", subcore_axis_name="s")
    run = pl.kernel(
        functools.partial(_sc_body, units1, units2),
        out_type=jax.ShapeDtypeStruct((2, 2, NP, H), jnp.float32),
        mesh=mesh,
        scratch_types=[
            pltpu.VMEM((2, NBUF, 3, C), jnp.int32),
            pltpu.VMEM((NBUF, C, H), jnp.float32),
            pltpu.VMEM_SHARED((NP, H), jnp.float32),
        ] + [pltpu.SemaphoreType.DMA] * (2 + 2 * NBUF),
        compiler_params=pltpu.CompilerParams(use_tc_tiling_on_sc=False,
                                             needs_layout_passes=False),
        name="h2gcn_spmm_sc",
    )
    out = run(xcat, icrw1, icrw2)
    return jnp.concatenate([out[0, 0, :N], out[0, 1, :N],
                            out[1, 0, :N], out[1, 1, :N]], axis=1)


# speedup vs baseline: 1.1803x; 1.1075x over previous
import functools

import jax
import jax.numpy as jnp
from jax import lax
from jax.experimental import pallas as pl
from jax.experimental.pallas import tpu as pltpu
from jax.experimental.pallas import tpu_sc as plsc

N = 10000
D = 128
H = D // 2
C = 256
SG = 1
NS = 16
NBUF = 4
NP = 10240
ROWS_PER_TILE = NP // NS
ROWS_PER_COPY = ROWS_PER_TILE // 5
UNIT_E = SG * C


def _prep_edges(edge_index, edge_weight, units_per_tile):
    e = edge_index.shape[1]
    ep = NS * units_per_tile * UNIT_E
    col = jnp.pad(edge_index[1].astype(jnp.int32), (0, ep - e))
    row = jnp.pad(edge_index[0].astype(jnp.int32), (0, ep - e))
    w = jnp.pad(edge_weight, (0, ep - e))
    nu = ep // UNIT_E
    col = col.reshape(nu, C)
    row = row.reshape(nu, C)
    wb = jax.lax.bitcast_convert_type(w, jnp.int32).reshape(nu, C)
    core0 = jnp.stack([col, row, wb], axis=1)
    core1 = jnp.stack([col + N, row, wb], axis=1)
    icrw = jnp.stack([core0, core1])
    return jnp.pad(icrw, ((0, 0), (0, 2 * NBUF), (0, 0), (0, 0)))


def _sc_body(units1, units2,
             xcat_h, icrw1_h, icrw2_h, out_h,
             cb, gx, out_sh, *sems):
    c = lax.axis_index("c")
    s = lax.axis_index("s")
    isems = list(sems[:2])
    gsems = list(sems[2:2 + NBUF])
    ssems = list(sems[2 + NBUF:2 + 2 * NBUF])

    base_row = s * ROWS_PER_TILE
    zv = jnp.zeros((16,), jnp.float32)

    def zero_accumulator():
        @pl.loop(0, ROWS_PER_COPY)
        def _zero(e):
            for j in range(H // 16):
                gx[0, e, pl.ds(j * 16, 16)] = zv
        for k in range(5):
            r = base_row + k * ROWS_PER_COPY
            pltpu.sync_copy(gx.at[0, pl.ds(0, ROWS_PER_COPY)],
                            out_sh.at[pl.ds(r, ROWS_PER_COPY)])

    def copy_out(which):
        for k in range(5):
            r = base_row + k * ROWS_PER_COPY
            pltpu.sync_copy(out_sh.at[pl.ds(r, ROWS_PER_COPY)],
                            out_h.at[which, c, pl.ds(r, ROWS_PER_COPY)])

    def do_edges(icrw_h, units_per_tile):
        ubase = s * units_per_tile
        n_groups = units_per_tile // NBUF

        def idx_prefetch(slot, g):
            base = ubase + g * NBUF
            pltpu.async_copy(icrw_h.at[c, pl.ds(base, NBUF)], cb.at[slot],
                             isems[slot])

        def idx_wait(slot):
            pltpu.make_async_copy(icrw_h.at[c, pl.ds(ubase, NBUF)],
                                  cb.at[slot], isems[slot]).wait()

        def gather(slot, b):
            pltpu.async_copy(xcat_h.at[cb.at[slot, b, 0]], gx.at[b], gsems[b])

        def gather_wait(b):
            pltpu.make_async_copy(xcat_h.at[cb.at[0, b, 0]], gx.at[b],
                                  gsems[b]).wait()

        def scatter(slot, b):
            pltpu.async_copy(gx.at[b], out_sh.at[cb.at[slot, b, 1]], ssems[b],
                             add=True)

        def scatter_wait(slot, b):
            pltpu.make_async_copy(gx.at[b], out_sh.at[cb.at[slot, b, 1]],
                                  ssems[b]).wait()

        def scale(slot, b):
            @plsc.parallel_loop(0, C // 16, unroll=2)
            def _scale(g16):
                wi = cb[slot, b, 2, pl.ds(g16 * 16, 16)]
                wv = plsc.bitcast(wi, jnp.float32)
                for i in range(16):
                    e = g16 * 16 + i
                    w = wv[i]
                    for j in range(H // 16):
                        gx[b, e, pl.ds(j * 16, 16)] = (
                            gx[b, e, pl.ds(j * 16, 16)] * w)

        idx_prefetch(0, 0)
        idx_wait(0)
        idx_prefetch(1, 1)
        for b in range(NBUF):
            gather(0, b)

        @pl.loop(0, n_groups // 2)
        def _grp(gi):
            g = gi * 2
            for ph in range(2):
                other = 1 - ph
                for b in range(NBUF):
                    gather_wait(b)
                    scale(ph, b)
                    scatter(ph, b)
                idx_wait(other)
                for b in range(NBUF):
                    scatter_wait(ph, b)
                    gather(other, b)
                idx_prefetch(ph, g + ph + 2)

        for b in range(NBUF):
            gather_wait(b)
        idx_wait(1)

    zero_accumulator()
    plsc.subcore_barrier()
    do_edges(icrw1_h, units1)
    plsc.subcore_barrier()
    copy_out(0)
    zero_accumulator()
    plsc.subcore_barrier()
    do_edges(icrw2_h, units2)
    plsc.subcore_barrier()
    copy_out(1)


@jax.jit
def kernel(x, edge_index, edge_weight, edge_index2, edge_weight2):
    e1 = edge_index.shape[1]
    e2 = edge_index2.shape[1]
    per = NS * UNIT_E * 2 * NBUF
    units1 = (-(-e1 // per) * per) // (NS * UNIT_E)
    units2 = (-(-e2 // per) * per) // (NS * UNIT_E)

    xcat = jnp.concatenate([x[:, :H], x[:, H:]], axis=0)
    icrw1 = _prep_edges(edge_index, edge_weight, units1)
    icrw2 = _prep_edges(edge_index2, edge_weight2, units2)

    mesh = plsc.VectorSubcoreMesh(core_axis_name="c", subcore_axis_name="s")
    run = pl.kernel(
        functools.partial(_sc_body, units1, units2),
        out_type=jax.ShapeDtypeStruct((2, 2, NP, H), jnp.float32),
        mesh=mesh,
        scratch_types=[
            pltpu.VMEM((2, NBUF, 3, C), jnp.int32),
            pltpu.VMEM((NBUF, C, H), jnp.float32),
            pltpu.VMEM_SHARED((NP, H), jnp.float32),
        ] + [pltpu.SemaphoreType.DMA] * (2 + 2 * NBUF),
        compiler_params=pltpu.CompilerParams(use_tc_tiling_on_sc=False,
                                             needs_layout_passes=False),
        name="h2gcn_spmm_sc",
    )
    out = run(xcat, icrw1, icrw2)
    return jnp.concatenate([out[0, 0, :N], out[0, 1, :N],
                            out[1, 0, :N], out[1, 1, :N]], axis=1)
